# baseline (speedup 1.0000x reference)
import functools

import jax
import jax.numpy as jnp
from jax import lax
from jax.experimental import pallas as pl
from jax.experimental.pallas import tpu as pltpu
from jax.experimental.pallas import tpu_sc as plsc

_B, _T, _D = 4, 4096, 2048
_G, _CG, _K = 16, 128, 16
_N = _B * _T
_TB = 256
_NW = 32
_TPW = _N // _NW
_CH = 8


def _sc_body(tpw, cap_hbm, a_hbm, b_hbm, out_hbm, buf_v, a_v, b_v, th_v, sem):
    wid = lax.axis_index("s") * 2 + lax.axis_index("c")
    base_t = wid * tpw
    pltpu.sync_copy(a_hbm, a_v)
    pltpu.sync_copy(b_hbm, b_v)
    bvec = b_v[...]

    def merge(x, y):
        z = jnp.maximum(x, jnp.flip(y, axis=0))
        return plsc.sort_key_val(z, z, descending=True)[0]

    def token_body(tl, _):
        tv = jnp.zeros((_CG // 8,), jnp.float32)
        gidx = lax.iota(jnp.int32, 16)
        for g in range(_G):
            vs = []
            for i in range(8):
                sv = (buf_v[tl, pl.ds(g * _CG + i * 16, 16)] * bvec
                      + a_v[pl.ds(g * _CG + i * 16, 16)])
                vs.append(plsc.sort_key_val(sv, sv, descending=True)[0])
            m = [merge(vs[0], vs[1]), merge(vs[2], vs[3]),
                 merge(vs[4], vs[5]), merge(vs[6], vs[7])]
            m = [merge(m[0], m[1]), merge(m[2], m[3])]
            top = merge(m[0], m[1])
            th = jnp.min(top)
            tv = jnp.where(gidx == g, jnp.full((16,), th), tv)
        th_v[pl.ds(tl * _G, _G)] = tv
        return 0

    def chunk_body(c, _):
        cp = pltpu.make_async_copy(
            cap_hbm.at[pl.ds(base_t + c * _CH, _CH)], buf_v, sem)
        cp.start()
        cp.wait()

        def tb(tl, carry):
            token_body(tl, carry)
            return 0

        lax.fori_loop(0, _CH, tb, 0)
        pltpu.sync_copy(
            th_v.at[pl.ds(0, _CH * _G)],
            out_hbm.at[pl.ds((base_t + c * _CH) * _G, _CH * _G)])
        return 0

    lax.fori_loop(0, tpw // _CH, chunk_body, 0)


def _sc_thresh(cap2, a, bvec):
    n = cap2.shape[0]
    mesh = plsc.VectorSubcoreMesh(core_axis_name="c", subcore_axis_name="s")
    k = pl.kernel(
        functools.partial(_sc_body, n // _NW),
        out_type=jax.ShapeDtypeStruct((n * _G,), jnp.float32),
        mesh=mesh,
        scratch_types=[
            pltpu.VMEM((_CH, _D), jnp.float32),
            pltpu.VMEM((_D,), jnp.float32),
            pltpu.VMEM((16,), jnp.float32),
            pltpu.VMEM((_CH * _G,), jnp.float32),
            pltpu.SemaphoreType.DMA,
        ],
        compiler_params=pltpu.CompilerParams(needs_layout_passes=False),
    )
    return k(cap2, a.reshape(_D), bvec)


def _tc_body(a_ref, b_ref, th_ref, x_ref, u_ref, cap_ref, o_ref):
    b = b_ref[0, 0]
    s = cap_ref[...] * b + a_ref[...]
    for gi in range(_G):
        sl = pl.ds(gi * _CG, _CG)
        sg = s[:, gi * _CG:(gi + 1) * _CG]
        th = th_ref[:, gi:gi + 1]
        gated = jnp.where(sg >= th, sg, jnp.float32(-1e9))
        gg = jax.nn.sigmoid(gated)
        gg = gg * gg
        o_ref[:, sl] = x_ref[:, sl] + gg * u_ref[:, sl]


@jax.jit
def kernel(x, u, cap, logits, logit_scale, cap_scale):
    x2 = x.reshape(_N, _D)
    u2 = u.reshape(_N, _D)
    cap2 = cap.reshape(_N, _D)
    a = (logit_scale * logits).astype(jnp.float32)
    bs = (logit_scale * cap_scale).astype(jnp.float32)
    bvec = jnp.full((16,), bs, jnp.float32)

    h = _N // 2

    def tc_half(th, xh, uh, ch):
        grid = (h // _TB,)
        blk = pl.BlockSpec((_TB, _D), lambda i: (i, 0))
        return pl.pallas_call(
            _tc_body,
            grid=grid,
            in_specs=[
                pl.BlockSpec((1, _D), lambda i: (0, 0)),
                pl.BlockSpec((1, 1), lambda i: (0, 0)),
                pl.BlockSpec((_TB, _G), lambda i: (i, 0)),
                blk,
                blk,
                blk,
            ],
            out_specs=blk,
            out_shape=jax.ShapeDtypeStruct((h, _D), jnp.float32),
            compiler_params=pltpu.CompilerParams(
                dimension_semantics=("parallel",)),
        )(a.reshape(1, _D), bs.reshape(1, 1), th, xh, uh, ch)

    th0 = _sc_thresh(cap2[:h], a, bvec).reshape(h, _G)
    th1 = _sc_thresh(cap2[h:], a, bvec).reshape(h, _G)
    out0 = tc_half(th0, x2[:h], u2[:h], cap2[:h])
    out1 = tc_half(th1, x2[h:], u2[h:], cap2[h:])
    out = jnp.concatenate([out0, out1], axis=0)
    return out.reshape(_B, _T, _D)


# speedup vs baseline: 1.3613x; 1.3613x over previous
import functools

import jax
import jax.numpy as jnp
from jax import lax
from jax.experimental import pallas as pl
from jax.experimental.pallas import tpu as pltpu
from jax.experimental.pallas import tpu_sc as plsc

_B, _T, _D = 4, 4096, 2048
_G, _CG, _K = 16, 128, 16
_N = _B * _T
_TB = 256
_NW = 32
_TPW = _N // _NW
_CH = 8


def _sc_body(cap_hbm, a_hbm, b_hbm, out_hbm, buf_v, buf2_v, a_v, b_v, th_v, sem, sem2):
    wid = lax.axis_index("s") * 2 + lax.axis_index("c")
    base_t = wid * _TPW
    pltpu.sync_copy(a_hbm, a_v)
    pltpu.sync_copy(b_hbm, b_v)
    bvec = b_v[...]

    def merge(x, y):
        z = jnp.maximum(x, jnp.flip(y, axis=0))
        return plsc.sort_key_val(z, z, descending=True)[0]

    def token_body(tl, buf, _):
        tv = jnp.zeros((_CG // 8,), jnp.float32)
        gidx = lax.iota(jnp.int32, 16)
        for g in range(_G):
            vs = []
            for i in range(8):
                sv = (buf[tl, pl.ds(g * _CG + i * 16, 16)] * bvec
                      + a_v[pl.ds(g * _CG + i * 16, 16)])
                vs.append(plsc.sort_key_val(sv, sv, descending=True)[0])
            m = [merge(vs[0], vs[1]), merge(vs[2], vs[3]),
                 merge(vs[4], vs[5]), merge(vs[6], vs[7])]
            m = [merge(m[0], m[1]), merge(m[2], m[3])]
            top = merge(m[0], m[1])
            th = jnp.min(top)
            tv = jnp.where(gidx == g, jnp.full((16,), th), tv)
        th_v[pl.ds(tl * _G, _G)] = tv
        return 0

    nchunks = _TPW // _CH

    def dma(c, buf, sem_):
        return pltpu.make_async_copy(
            cap_hbm.at[pl.ds(base_t + c * _CH, _CH)], buf, sem_)

    def compute_chunk(c, buf):
        def tb(tl, carry):
            token_body(tl, buf, carry)
            return 0

        lax.fori_loop(0, _CH, tb, 0)
        pltpu.sync_copy(
            th_v.at[pl.ds(0, _CH * _G)],
            out_hbm.at[pl.ds((base_t + c * _CH) * _G, _CH * _G)])

    dma(0, buf_v, sem).start()

    def pair_body(p, _):
        c0 = p * 2
        dma(c0, buf_v, sem).wait()
        dma(c0 + 1, buf2_v, sem2).start()
        compute_chunk(c0, buf_v)
        dma(c0 + 1, buf2_v, sem2).wait()

        @pl.when(c0 + 2 < nchunks)
        def _():
            dma(c0 + 2, buf_v, sem).start()

        compute_chunk(c0 + 1, buf2_v)
        return 0

    lax.fori_loop(0, nchunks // 2, pair_body, 0)


def _sc_thresh(cap2, a, bvec):
    mesh = plsc.VectorSubcoreMesh(core_axis_name="c", subcore_axis_name="s")
    k = pl.kernel(
        _sc_body,
        out_type=jax.ShapeDtypeStruct((_N * _G,), jnp.float32),
        mesh=mesh,
        scratch_types=[
            pltpu.VMEM((_CH, _D), jnp.float32),
            pltpu.VMEM((_CH, _D), jnp.float32),
            pltpu.VMEM((_D,), jnp.float32),
            pltpu.VMEM((16,), jnp.float32),
            pltpu.VMEM((_CH * _G,), jnp.float32),
            pltpu.SemaphoreType.DMA,
            pltpu.SemaphoreType.DMA,
        ],
        compiler_params=pltpu.CompilerParams(needs_layout_passes=False),
    )
    return k(cap2.reshape(_N * _D // _D, _D), a.reshape(_D), bvec)


def _tc_body(a_ref, b_ref, th_ref, x_ref, u_ref, cap_ref, o_ref):
    b = b_ref[0, 0]
    s = cap_ref[...] * b + a_ref[...]
    for gi in range(_G):
        sl = pl.ds(gi * _CG, _CG)
        sg = s[:, gi * _CG:(gi + 1) * _CG]
        th = th_ref[:, gi:gi + 1]
        gated = jnp.where(sg >= th, sg, jnp.float32(-1e9))
        gg = jax.nn.sigmoid(gated)
        gg = gg * gg
        o_ref[:, sl] = x_ref[:, sl] + gg * u_ref[:, sl]


@jax.jit
def kernel(x, u, cap, logits, logit_scale, cap_scale):
    x2 = x.reshape(_N, _D)
    u2 = u.reshape(_N, _D)
    cap2 = cap.reshape(_N, _D)
    a = (logit_scale * logits).astype(jnp.float32)
    bs = (logit_scale * cap_scale).astype(jnp.float32)
    bvec = jnp.full((16,), bs, jnp.float32)

    th = _sc_thresh(cap2, a, bvec).reshape(_N, _G)

    grid = (_N // _TB,)
    blk = pl.BlockSpec((_TB, _D), lambda i: (i, 0))
    out = pl.pallas_call(
        _tc_body,
        grid=grid,
        in_specs=[
            pl.BlockSpec((1, _D), lambda i: (0, 0)),
            pl.BlockSpec((1, 1), lambda i: (0, 0)),
            pl.BlockSpec((_TB, _G), lambda i: (i, 0)),
            blk,
            blk,
            blk,
        ],
        out_specs=blk,
        out_shape=jax.ShapeDtypeStruct((_N, _D), jnp.float32),
        compiler_params=pltpu.CompilerParams(
            dimension_semantics=("parallel",)),
    )(a.reshape(1, _D), bs.reshape(1, 1), th, x2, u2, cap2)
    return out.reshape(_B, _T, _D)
